# baseline (speedup 1.0000x reference)
import jax
import jax.numpy as jnp
from jax import lax
from jax.experimental import pallas as pl
from jax.experimental.pallas import tpu as pltpu
from jax.experimental.pallas import tpu_sc as plsc

_N = 10000
_E = 320000
_H = 128
_CHUNK = 128
_NTILES = 32
_NSUB = 16
_CPT = 80
_NCHUNKS = _NTILES * _CPT
_EPAD = _NCHUNKS * _CHUNK
_ROWS_PER_TILE = 632
_NPAD = _NSUB * _ROWS_PER_TILE
_EPS = 1e-5

_mesh = plsc.VectorSubcoreMesh(core_axis_name="c", subcore_axis_name="s")


def _seg_sum_body(h_hbm, src_hbm, dst_hbm, zeros_hbm, out_hbm,
                  shared, isrc, idst, rows0, rows1, sem0, sem1):
    c = lax.axis_index("c")
    s = lax.axis_index("s")
    wid = s * 2 + c
    base = wid * _CPT
    pltpu.sync_copy(zeros_hbm,
                    shared.at[pl.ds(s * _ROWS_PER_TILE, _ROWS_PER_TILE)])
    plsc.subcore_barrier()

    half = _CPT // 2
    npairs = half // 2
    for hh in range(2):
        pltpu.sync_copy(src_hbm.at[pl.ds(base + hh * half, half)], isrc)
        pltpu.sync_copy(dst_hbm.at[pl.ds(base + hh * half, half)], idst)
        pltpu.async_copy(h_hbm.at[isrc.at[0]], rows0, sem0)
        pltpu.async_copy(h_hbm.at[isrc.at[1]], rows1, sem1)

        def body(k, carry):
            a = 2 * k
            b = a + 1
            pltpu.make_async_copy(h_hbm.at[isrc.at[a]], rows0, sem0).wait()
            pltpu.sync_copy(rows0, shared.at[idst.at[a]], add=True)

            @pl.when(k < npairs - 1)
            def _():
                pltpu.async_copy(h_hbm.at[isrc.at[a + 2]], rows0, sem0)

            pltpu.make_async_copy(h_hbm.at[isrc.at[b]], rows1, sem1).wait()
            pltpu.sync_copy(rows1, shared.at[idst.at[b]], add=True)

            @pl.when(k < npairs - 1)
            def _():
                pltpu.async_copy(h_hbm.at[isrc.at[b + 2]], rows1, sem1)

            return carry

        lax.fori_loop(0, npairs, body, 0)
    plsc.subcore_barrier()
    pltpu.sync_copy(shared.at[pl.ds(s * _ROWS_PER_TILE, _ROWS_PER_TILE)],
                    out_hbm.at[c, pl.ds(s * _ROWS_PER_TILE, _ROWS_PER_TILE)])


_seg_sum = pl.kernel(
    _seg_sum_body,
    out_type=jax.ShapeDtypeStruct((2, _NPAD, _H), jnp.float32),
    mesh=_mesh,
    scratch_types=[
        pltpu.VMEM_SHARED((_NPAD, _H), jnp.float32),
        pltpu.VMEM((_CPT // 2, _CHUNK), jnp.int32),
        pltpu.VMEM((_CPT // 2, _CHUNK), jnp.int32),
        pltpu.VMEM((_CHUNK, _H), jnp.float32),
        pltpu.VMEM((_CHUNK, _H), jnp.float32),
        pltpu.SemaphoreType.DMA,
        pltpu.SemaphoreType.DMA,
    ],
)


def _sq_diff_acc(rows_s, rows_d, acc):
    def row_body(i, a):
        for t in range(_H // 16):
            va = rows_s[i, pl.ds(t * 16, 16)]
            vb = rows_d[i, pl.ds(t * 16, 16)]
            dv = va - vb
            a = a + dv * dv
        return a

    return lax.fori_loop(0, _CHUNK, row_body, acc)


def _dirichlet_body(h_hbm, src_hbm, dst_hbm, out_hbm,
                    isrc, idst, ra0, rb0, ra1, rb1, acc_v,
                    sa0, sb0, sa1, sb1):
    c = lax.axis_index("c")
    s = lax.axis_index("s")
    wid = s * 2 + c
    base = wid * _CPT
    pltpu.sync_copy(src_hbm.at[pl.ds(base, _CPT)], isrc)
    pltpu.sync_copy(dst_hbm.at[pl.ds(base, _CPT)], idst)
    pltpu.async_copy(h_hbm.at[isrc.at[0]], ra0, sa0)
    pltpu.async_copy(h_hbm.at[idst.at[0]], rb0, sb0)
    pltpu.async_copy(h_hbm.at[isrc.at[1]], ra1, sa1)
    pltpu.async_copy(h_hbm.at[idst.at[1]], rb1, sb1)

    def chunk_body(k, acc):
        a = 2 * k
        b = a + 1
        pltpu.make_async_copy(h_hbm.at[isrc.at[a]], ra0, sa0).wait()
        pltpu.make_async_copy(h_hbm.at[idst.at[a]], rb0, sb0).wait()
        acc = _sq_diff_acc(ra0, rb0, acc)

        @pl.when(k < _CPT // 2 - 1)
        def _():
            pltpu.async_copy(h_hbm.at[isrc.at[a + 2]], ra0, sa0)
            pltpu.async_copy(h_hbm.at[idst.at[a + 2]], rb0, sb0)

        pltpu.make_async_copy(h_hbm.at[isrc.at[b]], ra1, sa1).wait()
        pltpu.make_async_copy(h_hbm.at[idst.at[b]], rb1, sb1).wait()
        acc = _sq_diff_acc(ra1, rb1, acc)

        @pl.when(k < _CPT // 2 - 1)
        def _():
            pltpu.async_copy(h_hbm.at[isrc.at[b + 2]], ra1, sa1)
            pltpu.async_copy(h_hbm.at[idst.at[b + 2]], rb1, sb1)

        return acc

    acc = lax.fori_loop(0, _CPT // 2, chunk_body, jnp.zeros((16,), jnp.float32))
    acc_v[...] = acc
    pltpu.sync_copy(acc_v, out_hbm.at[pl.ds(wid * 16, 16)])


_dirichlet = pl.kernel(
    _dirichlet_body,
    out_type=jax.ShapeDtypeStruct((_NTILES * 16,), jnp.float32),
    mesh=_mesh,
    scratch_types=[
        pltpu.VMEM((_CPT, _CHUNK), jnp.int32),
        pltpu.VMEM((_CPT, _CHUNK), jnp.int32),
        pltpu.VMEM((_CHUNK, _H), jnp.float32),
        pltpu.VMEM((_CHUNK, _H), jnp.float32),
        pltpu.VMEM((_CHUNK, _H), jnp.float32),
        pltpu.VMEM((_CHUNK, _H), jnp.float32),
        pltpu.VMEM((16,), jnp.float32),
        pltpu.SemaphoreType.DMA,
        pltpu.SemaphoreType.DMA,
        pltpu.SemaphoreType.DMA,
        pltpu.SemaphoreType.DMA,
    ],
)


_BLK = 1000


def _bn_apply(y, g, bt, rm, rv):
    scale = g / jnp.sqrt(rv + _EPS)
    return y * scale + (bt - rm * scale)


def _pre_body(x_ref, w_ref, b_ref, o_ref):
    y = jnp.dot(x_ref[...], w_ref[...], preferred_element_type=jnp.float32)
    o_ref[...] = jnp.maximum(y + b_ref[...], 0.0)


def _pre_call(x, w, b2d):
    return pl.pallas_call(
        _pre_body,
        grid=(_N // _BLK,),
        in_specs=[
            pl.BlockSpec((_BLK, _H), lambda i: (i, 0)),
            pl.BlockSpec((_H, _H), lambda i: (0, 0)),
            pl.BlockSpec((1, _H), lambda i: (0, 0)),
        ],
        out_specs=pl.BlockSpec((_BLK, _H), lambda i: (i, 0)),
        out_shape=jax.ShapeDtypeStruct((_N, _H), jnp.float32),
    )(x, w, b2d)


def _gin_body(h_ref, agg_ref, w1_ref, f1_ref, w2_ref, f2_ref, o_ref):
    h = h_ref[...]
    m = h + agg_ref[0] + agg_ref[1]
    y = jnp.dot(m, w1_ref[...], preferred_element_type=jnp.float32)
    y = _bn_apply(y + f1_ref[0:1, :], f1_ref[1:2, :], f1_ref[2:3, :],
                  f1_ref[3:4, :], f1_ref[4:5, :])
    y = jnp.maximum(y, 0.0)
    y = jnp.dot(y, w2_ref[...], preferred_element_type=jnp.float32)
    y = _bn_apply(y + f2_ref[0:1, :], f2_ref[1:2, :], f2_ref[2:3, :],
                  f2_ref[3:4, :], f2_ref[4:5, :])
    o_ref[...] = h + jnp.maximum(y, 0.0)


def _gin_call(h, agg, p):
    f1 = jnp.stack([p["b1"], p["g1"], p["bt1"], p["rm1"], p["rv1"]])
    f2 = jnp.stack([p["b2"], p["g2"], p["bt2"], p["rm2"], p["rv2"]])
    return pl.pallas_call(
        _gin_body,
        grid=(_N // _BLK,),
        in_specs=[
            pl.BlockSpec((_BLK, _H), lambda i: (i, 0)),
            pl.BlockSpec((2, _BLK, _H), lambda i: (0, i, 0)),
            pl.BlockSpec((_H, _H), lambda i: (0, 0)),
            pl.BlockSpec((5, _H), lambda i: (0, 0)),
            pl.BlockSpec((_H, _H), lambda i: (0, 0)),
            pl.BlockSpec((5, _H), lambda i: (0, 0)),
        ],
        out_specs=pl.BlockSpec((_BLK, _H), lambda i: (i, 0)),
        out_shape=jax.ShapeDtypeStruct((_N, _H), jnp.float32),
    )(h, agg, p["W1"], f1, p["W2"], f2)


def _head_body(ctrl_ref, h_ref, parts_ref, w3_ref, b3_ref, bn_ref,
               w4_ref, b4_ref, o_ref, de_ref):
    i = ctrl_ref[0]
    hh = h_ref[pl.ds(i, 1), :]
    for t in range(3):
        y = jnp.dot(hh, w3_ref[t], preferred_element_type=jnp.float32)
        y = _bn_apply(y + b3_ref[t:t + 1, :], bn_ref[0:1, :], bn_ref[1:2, :],
                      bn_ref[2:3, :], bn_ref[3:4, :])
        hh = jnp.maximum(y, 0.0)
    o_ref[...] = jnp.sum(hh * w4_ref[...], axis=1, keepdims=True) + b4_ref[...]
    de_ref[...] = (0.5 * jnp.sum(parts_ref[...]) / _N).reshape(1, 1)


def _head_call(ctrl, h, parts, Wl1, bl1, Wl2, bl2, Wl3, bl3, Wl4, bl4,
               bn_g, bn_b, bn_rm, bn_rv):
    w3 = jnp.stack([Wl1, Wl2, Wl3])
    b3 = jnp.stack([bl1, bl2, bl3])
    bn = jnp.stack([bn_g, bn_b, bn_rm, bn_rv])
    w4 = Wl4[:, 0].reshape(1, _H)
    return pl.pallas_call(
        _head_body,
        in_specs=[
            pl.BlockSpec(memory_space=pltpu.SMEM),
            pl.BlockSpec((_N, _H), lambda: (0, 0)),
            pl.BlockSpec((4, _H), lambda: (0, 0)),
            pl.BlockSpec((3, _H, _H), lambda: (0, 0, 0)),
            pl.BlockSpec((3, _H), lambda: (0, 0)),
            pl.BlockSpec((4, _H), lambda: (0, 0)),
            pl.BlockSpec((1, _H), lambda: (0, 0)),
            pl.BlockSpec((1, 1), lambda: (0, 0)),
        ],
        out_specs=[
            pl.BlockSpec((1, 1), lambda: (0, 0)),
            pl.BlockSpec((1, 1), lambda: (0, 0)),
        ],
        out_shape=[
            jax.ShapeDtypeStruct((1, 1), jnp.float32),
            jax.ShapeDtypeStruct((1, 1), jnp.float32),
        ],
    )(ctrl, h, parts, w3, b3, bn, w4, bl4.reshape(1, 1))


def kernel(x, edge_index, batch, ctrl, W_pre, b_pre, gin_params,
           Wl1, bl1, Wl2, bl2, Wl3, bl3, Wl4, bl4,
           bn_g, bn_b, bn_rm, bn_rv):
    npad_e = _EPAD - _E
    src2 = jnp.concatenate(
        [edge_index[0], jnp.zeros((npad_e,), jnp.int32)]).reshape(_NCHUNKS, _CHUNK)
    dst_seg = jnp.concatenate(
        [edge_index[1], jnp.full((npad_e,), _N, jnp.int32)]).reshape(_NCHUNKS, _CHUNK)
    dst_dir = jnp.concatenate(
        [edge_index[1], jnp.zeros((npad_e,), jnp.int32)]).reshape(_NCHUNKS, _CHUNK)
    zeros = jnp.zeros((_ROWS_PER_TILE, _H), jnp.float32)

    h = _pre_call(x, W_pre, b_pre.reshape(1, _H))
    for p in gin_params:
        agg = _seg_sum(h, src2, dst_seg, zeros)
        h = _gin_call(h, agg, p)

    parts = _dirichlet(h, src2, dst_dir)
    o, de = _head_call(ctrl, h, parts.reshape(4, _H),
                       Wl1, bl1, Wl2, bl2, Wl3, bl3, Wl4, bl4,
                       bn_g, bn_b, bn_rm, bn_rv)
    return (o, o, de[0, 0])


# speedup vs baseline: 4.0506x; 4.0506x over previous
import jax
import jax.numpy as jnp
from jax import lax
from jax.experimental import pallas as pl
from jax.experimental.pallas import tpu as pltpu
from jax.experimental.pallas import tpu_sc as plsc

_N = 10000
_E = 320000
_H = 128
_CHUNK = 128
_NTILES = 32
_NSUB = 16
_CPT = 80
_NCHUNKS = _NTILES * _CPT
_EPAD = _NCHUNKS * _CHUNK
_ROWS_PER_TILE = 632
_NPAD = _NSUB * _ROWS_PER_TILE
_EPS = 1e-5

_mesh = plsc.VectorSubcoreMesh(core_axis_name="c", subcore_axis_name="s")


def _seg_sum_body(h_hbm, src_hbm, dst_hbm, zeros_hbm, out_hbm,
                  shared, isrc, idst, rows0, rows1, sem0, sem1):
    c = lax.axis_index("c")
    s = lax.axis_index("s")
    wid = s * 2 + c
    base = wid * _CPT
    pltpu.sync_copy(zeros_hbm,
                    shared.at[pl.ds(s * _ROWS_PER_TILE, _ROWS_PER_TILE)])
    plsc.subcore_barrier()

    half = _CPT // 2
    npairs = half // 2
    for hh in range(2):
        pltpu.sync_copy(src_hbm.at[pl.ds(base + hh * half, half)], isrc)
        pltpu.sync_copy(dst_hbm.at[pl.ds(base + hh * half, half)], idst)
        pltpu.async_copy(h_hbm.at[isrc.at[0]], rows0, sem0)
        pltpu.async_copy(h_hbm.at[isrc.at[1]], rows1, sem1)

        def body(k, carry):
            a = 2 * k
            b = a + 1
            pltpu.make_async_copy(h_hbm.at[isrc.at[a]], rows0, sem0).wait()
            pltpu.sync_copy(rows0, shared.at[idst.at[a]], add=True)

            @pl.when(k < npairs - 1)
            def _():
                pltpu.async_copy(h_hbm.at[isrc.at[a + 2]], rows0, sem0)

            pltpu.make_async_copy(h_hbm.at[isrc.at[b]], rows1, sem1).wait()
            pltpu.sync_copy(rows1, shared.at[idst.at[b]], add=True)

            @pl.when(k < npairs - 1)
            def _():
                pltpu.async_copy(h_hbm.at[isrc.at[b + 2]], rows1, sem1)

            return carry

        lax.fori_loop(0, npairs, body, 0)
    plsc.subcore_barrier()
    pltpu.sync_copy(shared.at[pl.ds(s * _ROWS_PER_TILE, _ROWS_PER_TILE)],
                    out_hbm.at[c, pl.ds(s * _ROWS_PER_TILE, _ROWS_PER_TILE)])


_seg_sum = pl.kernel(
    _seg_sum_body,
    out_type=jax.ShapeDtypeStruct((2, _NPAD, _H), jnp.float32),
    mesh=_mesh,
    scratch_types=[
        pltpu.VMEM_SHARED((_NPAD, _H), jnp.float32),
        pltpu.VMEM((_CPT // 2, _CHUNK), jnp.int32),
        pltpu.VMEM((_CPT // 2, _CHUNK), jnp.int32),
        pltpu.VMEM((_CHUNK, _H), jnp.float32),
        pltpu.VMEM((_CHUNK, _H), jnp.float32),
        pltpu.SemaphoreType.DMA,
        pltpu.SemaphoreType.DMA,
    ],
)


def _sq_diff_acc(rows_s, rows_d, acc):
    def row_body(i, a):
        for t in range(_H // 16):
            va = rows_s[i, pl.ds(t * 16, 16)]
            vb = rows_d[i, pl.ds(t * 16, 16)]
            dv = va - vb
            a = a + dv * dv
        return a

    return lax.fori_loop(0, _CHUNK, row_body, acc)


def _dirichlet_body(h_hbm, src_hbm, dst_hbm, out_hbm,
                    isrc, idst, ra0, rb0, ra1, rb1, acc_v,
                    sa0, sb0, sa1, sb1):
    c = lax.axis_index("c")
    s = lax.axis_index("s")
    wid = s * 2 + c
    base = wid * _CPT
    pltpu.sync_copy(src_hbm.at[pl.ds(base, _CPT)], isrc)
    pltpu.sync_copy(dst_hbm.at[pl.ds(base, _CPT)], idst)
    pltpu.async_copy(h_hbm.at[isrc.at[0]], ra0, sa0)
    pltpu.async_copy(h_hbm.at[idst.at[0]], rb0, sb0)
    pltpu.async_copy(h_hbm.at[isrc.at[1]], ra1, sa1)
    pltpu.async_copy(h_hbm.at[idst.at[1]], rb1, sb1)

    def chunk_body(k, acc):
        a = 2 * k
        b = a + 1
        pltpu.make_async_copy(h_hbm.at[isrc.at[a]], ra0, sa0).wait()
        pltpu.make_async_copy(h_hbm.at[idst.at[a]], rb0, sb0).wait()
        acc = _sq_diff_acc(ra0, rb0, acc)

        @pl.when(k < _CPT // 2 - 1)
        def _():
            pltpu.async_copy(h_hbm.at[isrc.at[a + 2]], ra0, sa0)
            pltpu.async_copy(h_hbm.at[idst.at[a + 2]], rb0, sb0)

        pltpu.make_async_copy(h_hbm.at[isrc.at[b]], ra1, sa1).wait()
        pltpu.make_async_copy(h_hbm.at[idst.at[b]], rb1, sb1).wait()
        acc = _sq_diff_acc(ra1, rb1, acc)

        @pl.when(k < _CPT // 2 - 1)
        def _():
            pltpu.async_copy(h_hbm.at[isrc.at[b + 2]], ra1, sa1)
            pltpu.async_copy(h_hbm.at[idst.at[b + 2]], rb1, sb1)

        return acc

    acc = lax.fori_loop(0, _CPT // 2, chunk_body, jnp.zeros((16,), jnp.float32))
    acc_v[...] = acc
    pltpu.sync_copy(acc_v, out_hbm.at[pl.ds(wid * 16, 16)])


_dirichlet = pl.kernel(
    _dirichlet_body,
    out_type=jax.ShapeDtypeStruct((_NTILES * 16,), jnp.float32),
    mesh=_mesh,
    scratch_types=[
        pltpu.VMEM((_CPT, _CHUNK), jnp.int32),
        pltpu.VMEM((_CPT, _CHUNK), jnp.int32),
        pltpu.VMEM((_CHUNK, _H), jnp.float32),
        pltpu.VMEM((_CHUNK, _H), jnp.float32),
        pltpu.VMEM((_CHUNK, _H), jnp.float32),
        pltpu.VMEM((_CHUNK, _H), jnp.float32),
        pltpu.VMEM((16,), jnp.float32),
        pltpu.SemaphoreType.DMA,
        pltpu.SemaphoreType.DMA,
        pltpu.SemaphoreType.DMA,
        pltpu.SemaphoreType.DMA,
    ],
)


_BLK = 1000


def _bn_apply(y, g, bt, rm, rv):
    scale = g / jnp.sqrt(rv + _EPS)
    return y * scale + (bt - rm * scale)


def _pre_body(x_ref, w_ref, b_ref, o_ref):
    y = jnp.dot(x_ref[...], w_ref[...], preferred_element_type=jnp.float32)
    o_ref[...] = jnp.maximum(y + b_ref[...], 0.0)


def _pre_call(x, w, b2d):
    return pl.pallas_call(
        _pre_body,
        grid=(_N // _BLK,),
        in_specs=[
            pl.BlockSpec((_BLK, _H), lambda i: (i, 0)),
            pl.BlockSpec((_H, _H), lambda i: (0, 0)),
            pl.BlockSpec((1, _H), lambda i: (0, 0)),
        ],
        out_specs=pl.BlockSpec((_BLK, _H), lambda i: (i, 0)),
        out_shape=jax.ShapeDtypeStruct((_N, _H), jnp.float32),
    )(x, w, b2d)


def _gin_body(h_ref, agg_ref, w1_ref, f1_ref, w2_ref, f2_ref, o_ref):
    h = h_ref[...]
    m = h + agg_ref[0] + agg_ref[1]
    y = jnp.dot(m, w1_ref[...], preferred_element_type=jnp.float32)
    y = _bn_apply(y + f1_ref[0:1, :], f1_ref[1:2, :], f1_ref[2:3, :],
                  f1_ref[3:4, :], f1_ref[4:5, :])
    y = jnp.maximum(y, 0.0)
    y = jnp.dot(y, w2_ref[...], preferred_element_type=jnp.float32)
    y = _bn_apply(y + f2_ref[0:1, :], f2_ref[1:2, :], f2_ref[2:3, :],
                  f2_ref[3:4, :], f2_ref[4:5, :])
    o_ref[...] = h + jnp.maximum(y, 0.0)


def _gin_call(h, agg, p):
    f1 = jnp.stack([p["b1"], p["g1"], p["bt1"], p["rm1"], p["rv1"]])
    f2 = jnp.stack([p["b2"], p["g2"], p["bt2"], p["rm2"], p["rv2"]])
    return pl.pallas_call(
        _gin_body,
        grid=(_N // _BLK,),
        in_specs=[
            pl.BlockSpec((_BLK, _H), lambda i: (i, 0)),
            pl.BlockSpec((2, _BLK, _H), lambda i: (0, i, 0)),
            pl.BlockSpec((_H, _H), lambda i: (0, 0)),
            pl.BlockSpec((5, _H), lambda i: (0, 0)),
            pl.BlockSpec((_H, _H), lambda i: (0, 0)),
            pl.BlockSpec((5, _H), lambda i: (0, 0)),
        ],
        out_specs=pl.BlockSpec((_BLK, _H), lambda i: (i, 0)),
        out_shape=jax.ShapeDtypeStruct((_N, _H), jnp.float32),
    )(h, agg, p["W1"], f1, p["W2"], f2)


def _head_body(ctrl_ref, h_ref, parts_ref, w3_ref, b3_ref, bn_ref,
               w4_ref, b4_ref, o_ref, de_ref):
    i = ctrl_ref[0]
    hh = h_ref[pl.ds(i, 1), :]
    for t in range(3):
        y = jnp.dot(hh, w3_ref[t], preferred_element_type=jnp.float32)
        y = _bn_apply(y + b3_ref[t:t + 1, :], bn_ref[0:1, :], bn_ref[1:2, :],
                      bn_ref[2:3, :], bn_ref[3:4, :])
        hh = jnp.maximum(y, 0.0)
    o_ref[...] = jnp.sum(hh * w4_ref[...], axis=1, keepdims=True) + b4_ref[...]
    de_ref[...] = (0.5 * jnp.sum(parts_ref[...]) / _N).reshape(1, 1)


def _head_call(ctrl, h, parts, Wl1, bl1, Wl2, bl2, Wl3, bl3, Wl4, bl4,
               bn_g, bn_b, bn_rm, bn_rv):
    w3 = jnp.stack([Wl1, Wl2, Wl3])
    b3 = jnp.stack([bl1, bl2, bl3])
    bn = jnp.stack([bn_g, bn_b, bn_rm, bn_rv])
    w4 = Wl4[:, 0].reshape(1, _H)
    return pl.pallas_call(
        _head_body,
        in_specs=[
            pl.BlockSpec(memory_space=pltpu.SMEM),
            pl.BlockSpec((_N, _H), lambda: (0, 0)),
            pl.BlockSpec((4, _H), lambda: (0, 0)),
            pl.BlockSpec((3, _H, _H), lambda: (0, 0, 0)),
            pl.BlockSpec((3, _H), lambda: (0, 0)),
            pl.BlockSpec((4, _H), lambda: (0, 0)),
            pl.BlockSpec((1, _H), lambda: (0, 0)),
            pl.BlockSpec((1, 1), lambda: (0, 0)),
        ],
        out_specs=[
            pl.BlockSpec((1, 1), lambda: (0, 0)),
            pl.BlockSpec((1, 1), lambda: (0, 0)),
        ],
        out_shape=[
            jax.ShapeDtypeStruct((1, 1), jnp.float32),
            jax.ShapeDtypeStruct((1, 1), jnp.float32),
        ],
    )(ctrl, h, parts, w3, b3, bn, w4, bl4.reshape(1, 1))


def kernel(x, edge_index, batch, ctrl, W_pre, b_pre, gin_params,
           Wl1, bl1, Wl2, bl2, Wl3, bl3, Wl4, bl4,
           bn_g, bn_b, bn_rm, bn_rv):
    npad_e = _EPAD - _E
    pidx = jnp.arange(npad_e, dtype=jnp.int32)
    src_pad = pidx % _N
    src2 = jnp.concatenate(
        [edge_index[0], src_pad]).reshape(_NCHUNKS, _CHUNK)
    dst_seg = jnp.concatenate(
        [edge_index[1], _N + pidx % (_NPAD - _N)]).reshape(_NCHUNKS, _CHUNK)
    dst_dir = jnp.concatenate(
        [edge_index[1], src_pad]).reshape(_NCHUNKS, _CHUNK)
    zeros = jnp.zeros((_ROWS_PER_TILE, _H), jnp.float32)

    h = _pre_call(x, W_pre, b_pre.reshape(1, _H))
    for p in gin_params:
        agg = _seg_sum(h, src2, dst_seg, zeros)
        h = _gin_call(h, agg, p)

    parts = _dirichlet(h, src2, dst_dir)
    o, de = _head_call(ctrl, h, parts.reshape(4, _H),
                       Wl1, bl1, Wl2, bl2, Wl3, bl3, Wl4, bl4,
                       bn_g, bn_b, bn_rm, bn_rv)
    return (o, o, de[0, 0])


# speedup vs baseline: 4.1376x; 1.0215x over previous
import jax
import jax.numpy as jnp
from jax import lax
from jax.experimental import pallas as pl
from jax.experimental.pallas import tpu as pltpu
from jax.experimental.pallas import tpu_sc as plsc

_N = 10000
_E = 320000
_H = 128
_CHUNK = 128
_NTILES = 32
_NSUB = 16
_CPT = 80
_NCHUNKS = _NTILES * _CPT
_EPAD = _NCHUNKS * _CHUNK
_ROWS_PER_TILE = 632
_NPAD = _NSUB * _ROWS_PER_TILE
_EPS = 1e-5

_mesh = plsc.VectorSubcoreMesh(core_axis_name="c", subcore_axis_name="s")


def _seg_sum_body(h_hbm, src_hbm, dst_hbm, zeros_hbm, out_hbm,
                  shared, isrc, idst, rows0, rows1, sem0, sem1):
    c = lax.axis_index("c")
    s = lax.axis_index("s")
    wid = s * 2 + c
    base = wid * _CPT
    pltpu.sync_copy(zeros_hbm,
                    shared.at[pl.ds(s * _ROWS_PER_TILE, _ROWS_PER_TILE)])
    plsc.subcore_barrier()

    half = _CPT // 2
    npairs = half // 2
    for hh in range(2):
        pltpu.sync_copy(src_hbm.at[pl.ds(base + hh * half, half)], isrc)
        pltpu.sync_copy(dst_hbm.at[pl.ds(base + hh * half, half)], idst)
        pltpu.async_copy(h_hbm.at[isrc.at[0]], rows0, sem0)
        pltpu.async_copy(h_hbm.at[isrc.at[1]], rows1, sem1)

        def body(k, carry):
            a = 2 * k
            b = a + 1
            pltpu.make_async_copy(h_hbm.at[isrc.at[a]], rows0, sem0).wait()
            pltpu.sync_copy(rows0, shared.at[idst.at[a]], add=True)

            @pl.when(k < npairs - 1)
            def _():
                pltpu.async_copy(h_hbm.at[isrc.at[a + 2]], rows0, sem0)

            pltpu.make_async_copy(h_hbm.at[isrc.at[b]], rows1, sem1).wait()
            pltpu.sync_copy(rows1, shared.at[idst.at[b]], add=True)

            @pl.when(k < npairs - 1)
            def _():
                pltpu.async_copy(h_hbm.at[isrc.at[b + 2]], rows1, sem1)

            return carry

        lax.fori_loop(0, npairs, body, 0)
    plsc.subcore_barrier()
    pltpu.sync_copy(shared.at[pl.ds(s * _ROWS_PER_TILE, _ROWS_PER_TILE)],
                    out_hbm.at[c, pl.ds(s * _ROWS_PER_TILE, _ROWS_PER_TILE)])


_seg_sum = pl.kernel(
    _seg_sum_body,
    out_type=jax.ShapeDtypeStruct((2, _NPAD, _H), jnp.float32),
    mesh=_mesh,
    scratch_types=[
        pltpu.VMEM_SHARED((_NPAD, _H), jnp.float32),
        pltpu.VMEM((_CPT // 2, _CHUNK), jnp.int32),
        pltpu.VMEM((_CPT // 2, _CHUNK), jnp.int32),
        pltpu.VMEM((_CHUNK, _H), jnp.float32),
        pltpu.VMEM((_CHUNK, _H), jnp.float32),
        pltpu.SemaphoreType.DMA,
        pltpu.SemaphoreType.DMA,
    ],
)


def _sq_diff_acc(rows_s, rows_d, acc):
    def row_body(i, a):
        for r in range(4):
            for t in range(_H // 16):
                va = rows_s[4 * i + r, pl.ds(t * 16, 16)]
                vb = rows_d[4 * i + r, pl.ds(t * 16, 16)]
                dv = va - vb
                a = a + dv * dv
        return a

    return lax.fori_loop(0, _CHUNK // 4, row_body, acc)


def _dirichlet_body(h_hbm, src_hbm, dst_hbm, out_hbm,
                    isrc, idst, ra0, rb0, ra1, rb1, acc_v,
                    sa0, sb0, sa1, sb1):
    c = lax.axis_index("c")
    s = lax.axis_index("s")
    wid = s * 2 + c
    base = wid * _CPT
    pltpu.sync_copy(src_hbm.at[pl.ds(base, _CPT)], isrc)
    pltpu.sync_copy(dst_hbm.at[pl.ds(base, _CPT)], idst)
    pltpu.async_copy(h_hbm.at[isrc.at[0]], ra0, sa0)
    pltpu.async_copy(h_hbm.at[idst.at[0]], rb0, sb0)
    pltpu.async_copy(h_hbm.at[isrc.at[1]], ra1, sa1)
    pltpu.async_copy(h_hbm.at[idst.at[1]], rb1, sb1)

    def chunk_body(k, acc):
        a = 2 * k
        b = a + 1
        pltpu.make_async_copy(h_hbm.at[isrc.at[a]], ra0, sa0).wait()
        pltpu.make_async_copy(h_hbm.at[idst.at[a]], rb0, sb0).wait()
        acc = _sq_diff_acc(ra0, rb0, acc)

        @pl.when(k < _CPT // 2 - 1)
        def _():
            pltpu.async_copy(h_hbm.at[isrc.at[a + 2]], ra0, sa0)
            pltpu.async_copy(h_hbm.at[idst.at[a + 2]], rb0, sb0)

        pltpu.make_async_copy(h_hbm.at[isrc.at[b]], ra1, sa1).wait()
        pltpu.make_async_copy(h_hbm.at[idst.at[b]], rb1, sb1).wait()
        acc = _sq_diff_acc(ra1, rb1, acc)

        @pl.when(k < _CPT // 2 - 1)
        def _():
            pltpu.async_copy(h_hbm.at[isrc.at[b + 2]], ra1, sa1)
            pltpu.async_copy(h_hbm.at[idst.at[b + 2]], rb1, sb1)

        return acc

    acc = lax.fori_loop(0, _CPT // 2, chunk_body, jnp.zeros((16,), jnp.float32))
    acc_v[...] = acc
    pltpu.sync_copy(acc_v, out_hbm.at[pl.ds(wid * 16, 16)])


_dirichlet = pl.kernel(
    _dirichlet_body,
    out_type=jax.ShapeDtypeStruct((_NTILES * 16,), jnp.float32),
    mesh=_mesh,
    scratch_types=[
        pltpu.VMEM((_CPT, _CHUNK), jnp.int32),
        pltpu.VMEM((_CPT, _CHUNK), jnp.int32),
        pltpu.VMEM((_CHUNK, _H), jnp.float32),
        pltpu.VMEM((_CHUNK, _H), jnp.float32),
        pltpu.VMEM((_CHUNK, _H), jnp.float32),
        pltpu.VMEM((_CHUNK, _H), jnp.float32),
        pltpu.VMEM((16,), jnp.float32),
        pltpu.SemaphoreType.DMA,
        pltpu.SemaphoreType.DMA,
        pltpu.SemaphoreType.DMA,
        pltpu.SemaphoreType.DMA,
    ],
)


_BLK = 2000


def _bn_apply(y, g, bt, rm, rv):
    scale = g / jnp.sqrt(rv + _EPS)
    return y * scale + (bt - rm * scale)


def _pre_body(x_ref, w_ref, b_ref, o_ref):
    y = jnp.dot(x_ref[...], w_ref[...], preferred_element_type=jnp.float32)
    o_ref[...] = jnp.maximum(y + b_ref[...], 0.0)


def _pre_call(x, w, b2d):
    return pl.pallas_call(
        _pre_body,
        grid=(_N // _BLK,),
        in_specs=[
            pl.BlockSpec((_BLK, _H), lambda i: (i, 0)),
            pl.BlockSpec((_H, _H), lambda i: (0, 0)),
            pl.BlockSpec((1, _H), lambda i: (0, 0)),
        ],
        out_specs=pl.BlockSpec((_BLK, _H), lambda i: (i, 0)),
        out_shape=jax.ShapeDtypeStruct((_N, _H), jnp.float32),
    )(x, w, b2d)


def _gin_body(h_ref, agg_ref, w1_ref, f1_ref, w2_ref, f2_ref, o_ref):
    h = h_ref[...]
    m = h + agg_ref[0] + agg_ref[1]
    y = jnp.dot(m, w1_ref[...], preferred_element_type=jnp.float32)
    y = _bn_apply(y + f1_ref[0:1, :], f1_ref[1:2, :], f1_ref[2:3, :],
                  f1_ref[3:4, :], f1_ref[4:5, :])
    y = jnp.maximum(y, 0.0)
    y = jnp.dot(y, w2_ref[...], preferred_element_type=jnp.float32)
    y = _bn_apply(y + f2_ref[0:1, :], f2_ref[1:2, :], f2_ref[2:3, :],
                  f2_ref[3:4, :], f2_ref[4:5, :])
    o_ref[...] = h + jnp.maximum(y, 0.0)


def _gin_call(h, agg, p):
    f1 = jnp.stack([p["b1"], p["g1"], p["bt1"], p["rm1"], p["rv1"]])
    f2 = jnp.stack([p["b2"], p["g2"], p["bt2"], p["rm2"], p["rv2"]])
    return pl.pallas_call(
        _gin_body,
        grid=(_N // _BLK,),
        in_specs=[
            pl.BlockSpec((_BLK, _H), lambda i: (i, 0)),
            pl.BlockSpec((2, _BLK, _H), lambda i: (0, i, 0)),
            pl.BlockSpec((_H, _H), lambda i: (0, 0)),
            pl.BlockSpec((5, _H), lambda i: (0, 0)),
            pl.BlockSpec((_H, _H), lambda i: (0, 0)),
            pl.BlockSpec((5, _H), lambda i: (0, 0)),
        ],
        out_specs=pl.BlockSpec((_BLK, _H), lambda i: (i, 0)),
        out_shape=jax.ShapeDtypeStruct((_N, _H), jnp.float32),
    )(h, agg, p["W1"], f1, p["W2"], f2)


def _head_body(ctrl_ref, h_ref, parts_ref, w3_ref, b3_ref, bn_ref,
               w4_ref, b4_ref, o_ref, de_ref):
    i = ctrl_ref[0]
    hh = h_ref[pl.ds(i, 1), :]
    for t in range(3):
        y = jnp.dot(hh, w3_ref[t], preferred_element_type=jnp.float32)
        y = _bn_apply(y + b3_ref[t:t + 1, :], bn_ref[0:1, :], bn_ref[1:2, :],
                      bn_ref[2:3, :], bn_ref[3:4, :])
        hh = jnp.maximum(y, 0.0)
    o_ref[...] = jnp.sum(hh * w4_ref[...], axis=1, keepdims=True) + b4_ref[...]
    de_ref[...] = (0.5 * jnp.sum(parts_ref[...]) / _N).reshape(1, 1)


def _head_call(ctrl, h, parts, Wl1, bl1, Wl2, bl2, Wl3, bl3, Wl4, bl4,
               bn_g, bn_b, bn_rm, bn_rv):
    w3 = jnp.stack([Wl1, Wl2, Wl3])
    b3 = jnp.stack([bl1, bl2, bl3])
    bn = jnp.stack([bn_g, bn_b, bn_rm, bn_rv])
    w4 = Wl4[:, 0].reshape(1, _H)
    return pl.pallas_call(
        _head_body,
        in_specs=[
            pl.BlockSpec(memory_space=pltpu.SMEM),
            pl.BlockSpec((_N, _H), lambda: (0, 0)),
            pl.BlockSpec((4, _H), lambda: (0, 0)),
            pl.BlockSpec((3, _H, _H), lambda: (0, 0, 0)),
            pl.BlockSpec((3, _H), lambda: (0, 0)),
            pl.BlockSpec((4, _H), lambda: (0, 0)),
            pl.BlockSpec((1, _H), lambda: (0, 0)),
            pl.BlockSpec((1, 1), lambda: (0, 0)),
        ],
        out_specs=[
            pl.BlockSpec((1, 1), lambda: (0, 0)),
            pl.BlockSpec((1, 1), lambda: (0, 0)),
        ],
        out_shape=[
            jax.ShapeDtypeStruct((1, 1), jnp.float32),
            jax.ShapeDtypeStruct((1, 1), jnp.float32),
        ],
    )(ctrl, h, parts, w3, b3, bn, w4, bl4.reshape(1, 1))


def kernel(x, edge_index, batch, ctrl, W_pre, b_pre, gin_params,
           Wl1, bl1, Wl2, bl2, Wl3, bl3, Wl4, bl4,
           bn_g, bn_b, bn_rm, bn_rv):
    npad_e = _EPAD - _E
    pidx = jnp.arange(npad_e, dtype=jnp.int32)
    src_pad = pidx % _N
    src2 = jnp.concatenate(
        [edge_index[0], src_pad]).reshape(_NCHUNKS, _CHUNK)
    dst_seg = jnp.concatenate(
        [edge_index[1], _N + pidx % (_NPAD - _N)]).reshape(_NCHUNKS, _CHUNK)
    dst_dir = jnp.concatenate(
        [edge_index[1], src_pad]).reshape(_NCHUNKS, _CHUNK)
    zeros = jnp.zeros((_ROWS_PER_TILE, _H), jnp.float32)

    h = _pre_call(x, W_pre, b_pre.reshape(1, _H))
    for p in gin_params:
        agg = _seg_sum(h, src2, dst_seg, zeros)
        h = _gin_call(h, agg, p)

    parts = _dirichlet(h, src2, dst_dir)
    o, de = _head_call(ctrl, h, parts.reshape(4, _H),
                       Wl1, bl1, Wl2, bl2, Wl3, bl3, Wl4, bl4,
                       bn_g, bn_b, bn_rm, bn_rv)
    return (o, o, de[0, 0])


# speedup vs baseline: 4.2468x; 1.0264x over previous
import jax
import jax.numpy as jnp
from jax import lax
from jax.experimental import pallas as pl
from jax.experimental.pallas import tpu as pltpu
from jax.experimental.pallas import tpu_sc as plsc

_N = 10000
_E = 320000
_H = 128
_CHUNK = 128
_NTILES = 32
_NSUB = 16
_CPT = 80
_NCHUNKS = _NTILES * _CPT
_EPAD = _NCHUNKS * _CHUNK
_ROWS_PER_TILE = 632
_NPAD = _NSUB * _ROWS_PER_TILE
_EPS = 1e-5

_mesh = plsc.VectorSubcoreMesh(core_axis_name="c", subcore_axis_name="s")


def _seg_sum_body(h_hbm, src_hbm, dst_hbm, zeros_hbm, out_hbm,
                  shared, isrc, idst, rows0, rows1, sem0, sem1):
    c = lax.axis_index("c")
    s = lax.axis_index("s")
    wid = s * 2 + c
    base = wid * _CPT
    pltpu.sync_copy(zeros_hbm,
                    shared.at[pl.ds(s * _ROWS_PER_TILE, _ROWS_PER_TILE)])
    plsc.subcore_barrier()

    half = _CPT // 2
    npairs = half // 2
    for hh in range(2):
        pltpu.sync_copy(src_hbm.at[pl.ds(base + hh * half, half)], isrc)
        pltpu.sync_copy(dst_hbm.at[pl.ds(base + hh * half, half)], idst)
        pltpu.async_copy(h_hbm.at[isrc.at[0]], rows0, sem0)
        pltpu.async_copy(h_hbm.at[isrc.at[1]], rows1, sem1)

        def body(k, carry):
            a = 2 * k
            b = a + 1
            pltpu.make_async_copy(h_hbm.at[isrc.at[a]], rows0, sem0).wait()
            pltpu.sync_copy(rows0, shared.at[idst.at[a]], add=True)

            @pl.when(k < npairs - 1)
            def _():
                pltpu.async_copy(h_hbm.at[isrc.at[a + 2]], rows0, sem0)

            pltpu.make_async_copy(h_hbm.at[isrc.at[b]], rows1, sem1).wait()
            pltpu.sync_copy(rows1, shared.at[idst.at[b]], add=True)

            @pl.when(k < npairs - 1)
            def _():
                pltpu.async_copy(h_hbm.at[isrc.at[b + 2]], rows1, sem1)

            return carry

        lax.fori_loop(0, npairs, body, 0)
    plsc.subcore_barrier()
    pltpu.sync_copy(shared.at[pl.ds(s * _ROWS_PER_TILE, _ROWS_PER_TILE)],
                    out_hbm.at[c, pl.ds(s * _ROWS_PER_TILE, _ROWS_PER_TILE)])


_seg_sum = pl.kernel(
    _seg_sum_body,
    out_type=jax.ShapeDtypeStruct((2, _NPAD, _H), jnp.float32),
    mesh=_mesh,
    scratch_types=[
        pltpu.VMEM_SHARED((_NPAD, _H), jnp.float32),
        pltpu.VMEM((_CPT // 2, _CHUNK), jnp.int32),
        pltpu.VMEM((_CPT // 2, _CHUNK), jnp.int32),
        pltpu.VMEM((_CHUNK, _H), jnp.float32),
        pltpu.VMEM((_CHUNK, _H), jnp.float32),
        pltpu.SemaphoreType.DMA,
        pltpu.SemaphoreType.DMA,
    ],
)


def _sq_diff_acc(rows_s, rows_d, acc):
    def row_body(i, a):
        for r in range(4):
            for t in range(_H // 16):
                va = rows_s[4 * i + r, pl.ds(t * 16, 16)]
                vb = rows_d[4 * i + r, pl.ds(t * 16, 16)]
                dv = va - vb
                a = a + dv * dv
        return a

    return lax.fori_loop(0, _CHUNK // 4, row_body, acc)


def _dirichlet_body(h_hbm, src_hbm, dst_hbm, out_hbm,
                    isrc, idst, ra0, rb0, ra1, rb1, ra2, rb2, acc_v,
                    sa0, sb0, sa1, sb1, sa2, sb2):
    c = lax.axis_index("c")
    s = lax.axis_index("s")
    wid = s * 2 + c
    base = wid * _CPT
    pltpu.sync_copy(src_hbm.at[pl.ds(base, _CPT)], isrc)
    pltpu.sync_copy(dst_hbm.at[pl.ds(base, _CPT)], idst)
    pltpu.async_copy(h_hbm.at[isrc.at[0]], ra0, sa0)
    pltpu.async_copy(h_hbm.at[idst.at[0]], rb0, sb0)
    pltpu.async_copy(h_hbm.at[isrc.at[1]], ra1, sa1)
    pltpu.async_copy(h_hbm.at[idst.at[1]], rb1, sb1)

    def chunk_body(k, acc):
        i0 = 3 * k
        i1 = i0 + 1
        i2 = i0 + 2
        pltpu.async_copy(h_hbm.at[isrc.at[i2]], ra2, sa2)
        pltpu.async_copy(h_hbm.at[idst.at[i2]], rb2, sb2)
        pltpu.make_async_copy(h_hbm.at[isrc.at[i0]], ra0, sa0).wait()
        pltpu.make_async_copy(h_hbm.at[idst.at[i0]], rb0, sb0).wait()
        acc = _sq_diff_acc(ra0, rb0, acc)
        pltpu.async_copy(h_hbm.at[isrc.at[i0 + 3]], ra0, sa0)
        pltpu.async_copy(h_hbm.at[idst.at[i0 + 3]], rb0, sb0)
        pltpu.make_async_copy(h_hbm.at[isrc.at[i1]], ra1, sa1).wait()
        pltpu.make_async_copy(h_hbm.at[idst.at[i1]], rb1, sb1).wait()
        acc = _sq_diff_acc(ra1, rb1, acc)
        pltpu.async_copy(h_hbm.at[isrc.at[i1 + 3]], ra1, sa1)
        pltpu.async_copy(h_hbm.at[idst.at[i1 + 3]], rb1, sb1)
        pltpu.make_async_copy(h_hbm.at[isrc.at[i2]], ra2, sa2).wait()
        pltpu.make_async_copy(h_hbm.at[idst.at[i2]], rb2, sb2).wait()
        acc = _sq_diff_acc(ra2, rb2, acc)
        return acc

    acc = lax.fori_loop(0, (_CPT - 2) // 3, chunk_body,
                        jnp.zeros((16,), jnp.float32))
    pltpu.make_async_copy(h_hbm.at[isrc.at[_CPT - 2]], ra0, sa0).wait()
    pltpu.make_async_copy(h_hbm.at[idst.at[_CPT - 2]], rb0, sb0).wait()
    acc = _sq_diff_acc(ra0, rb0, acc)
    pltpu.make_async_copy(h_hbm.at[isrc.at[_CPT - 1]], ra1, sa1).wait()
    pltpu.make_async_copy(h_hbm.at[idst.at[_CPT - 1]], rb1, sb1).wait()
    acc = _sq_diff_acc(ra1, rb1, acc)
    acc_v[...] = acc
    pltpu.sync_copy(acc_v, out_hbm.at[pl.ds(wid * 16, 16)])


_dirichlet = pl.kernel(
    _dirichlet_body,
    out_type=jax.ShapeDtypeStruct((_NTILES * 16,), jnp.float32),
    mesh=_mesh,
    scratch_types=[
        pltpu.VMEM((_CPT, _CHUNK), jnp.int32),
        pltpu.VMEM((_CPT, _CHUNK), jnp.int32),
        pltpu.VMEM((_CHUNK, _H), jnp.float32),
        pltpu.VMEM((_CHUNK, _H), jnp.float32),
        pltpu.VMEM((_CHUNK, _H), jnp.float32),
        pltpu.VMEM((_CHUNK, _H), jnp.float32),
        pltpu.VMEM((_CHUNK, _H), jnp.float32),
        pltpu.VMEM((_CHUNK, _H), jnp.float32),
        pltpu.VMEM((16,), jnp.float32),
        pltpu.SemaphoreType.DMA,
        pltpu.SemaphoreType.DMA,
        pltpu.SemaphoreType.DMA,
        pltpu.SemaphoreType.DMA,
        pltpu.SemaphoreType.DMA,
        pltpu.SemaphoreType.DMA,
    ],
)


_BLK = 2000


def _bn_apply(y, g, bt, rm, rv):
    scale = g / jnp.sqrt(rv + _EPS)
    return y * scale + (bt - rm * scale)


def _pre_body(x_ref, w_ref, b_ref, o_ref):
    y = jnp.dot(x_ref[...], w_ref[...], preferred_element_type=jnp.float32)
    o_ref[...] = jnp.maximum(y + b_ref[...], 0.0)


def _pre_call(x, w, b2d):
    return pl.pallas_call(
        _pre_body,
        grid=(_N // _BLK,),
        in_specs=[
            pl.BlockSpec((_BLK, _H), lambda i: (i, 0)),
            pl.BlockSpec((_H, _H), lambda i: (0, 0)),
            pl.BlockSpec((1, _H), lambda i: (0, 0)),
        ],
        out_specs=pl.BlockSpec((_BLK, _H), lambda i: (i, 0)),
        out_shape=jax.ShapeDtypeStruct((_N, _H), jnp.float32),
    )(x, w, b2d)


def _gin_body(h_ref, agg_ref, w1_ref, f1_ref, w2_ref, f2_ref, o_ref):
    h = h_ref[...]
    m = h + agg_ref[0] + agg_ref[1]
    y = jnp.dot(m, w1_ref[...], preferred_element_type=jnp.float32)
    y = _bn_apply(y + f1_ref[0:1, :], f1_ref[1:2, :], f1_ref[2:3, :],
                  f1_ref[3:4, :], f1_ref[4:5, :])
    y = jnp.maximum(y, 0.0)
    y = jnp.dot(y, w2_ref[...], preferred_element_type=jnp.float32)
    y = _bn_apply(y + f2_ref[0:1, :], f2_ref[1:2, :], f2_ref[2:3, :],
                  f2_ref[3:4, :], f2_ref[4:5, :])
    o_ref[...] = h + jnp.maximum(y, 0.0)


def _gin_call(h, agg, p):
    f1 = jnp.stack([p["b1"], p["g1"], p["bt1"], p["rm1"], p["rv1"]])
    f2 = jnp.stack([p["b2"], p["g2"], p["bt2"], p["rm2"], p["rv2"]])
    return pl.pallas_call(
        _gin_body,
        grid=(_N // _BLK,),
        in_specs=[
            pl.BlockSpec((_BLK, _H), lambda i: (i, 0)),
            pl.BlockSpec((2, _BLK, _H), lambda i: (0, i, 0)),
            pl.BlockSpec((_H, _H), lambda i: (0, 0)),
            pl.BlockSpec((5, _H), lambda i: (0, 0)),
            pl.BlockSpec((_H, _H), lambda i: (0, 0)),
            pl.BlockSpec((5, _H), lambda i: (0, 0)),
        ],
        out_specs=pl.BlockSpec((_BLK, _H), lambda i: (i, 0)),
        out_shape=jax.ShapeDtypeStruct((_N, _H), jnp.float32),
    )(h, agg, p["W1"], f1, p["W2"], f2)


def _head_body(ctrl_ref, h_ref, parts_ref, w3_ref, b3_ref, bn_ref,
               w4_ref, b4_ref, o_ref, de_ref):
    i = ctrl_ref[0]
    hh = h_ref[pl.ds(i, 1), :]
    for t in range(3):
        y = jnp.dot(hh, w3_ref[t], preferred_element_type=jnp.float32)
        y = _bn_apply(y + b3_ref[t:t + 1, :], bn_ref[0:1, :], bn_ref[1:2, :],
                      bn_ref[2:3, :], bn_ref[3:4, :])
        hh = jnp.maximum(y, 0.0)
    o_ref[...] = jnp.sum(hh * w4_ref[...], axis=1, keepdims=True) + b4_ref[...]
    de_ref[...] = (0.5 * jnp.sum(parts_ref[...]) / _N).reshape(1, 1)


def _head_call(ctrl, h, parts, Wl1, bl1, Wl2, bl2, Wl3, bl3, Wl4, bl4,
               bn_g, bn_b, bn_rm, bn_rv):
    w3 = jnp.stack([Wl1, Wl2, Wl3])
    b3 = jnp.stack([bl1, bl2, bl3])
    bn = jnp.stack([bn_g, bn_b, bn_rm, bn_rv])
    w4 = Wl4[:, 0].reshape(1, _H)
    return pl.pallas_call(
        _head_body,
        in_specs=[
            pl.BlockSpec(memory_space=pltpu.SMEM),
            pl.BlockSpec((_N, _H), lambda: (0, 0)),
            pl.BlockSpec((4, _H), lambda: (0, 0)),
            pl.BlockSpec((3, _H, _H), lambda: (0, 0, 0)),
            pl.BlockSpec((3, _H), lambda: (0, 0)),
            pl.BlockSpec((4, _H), lambda: (0, 0)),
            pl.BlockSpec((1, _H), lambda: (0, 0)),
            pl.BlockSpec((1, 1), lambda: (0, 0)),
        ],
        out_specs=[
            pl.BlockSpec((1, 1), lambda: (0, 0)),
            pl.BlockSpec((1, 1), lambda: (0, 0)),
        ],
        out_shape=[
            jax.ShapeDtypeStruct((1, 1), jnp.float32),
            jax.ShapeDtypeStruct((1, 1), jnp.float32),
        ],
    )(ctrl, h, parts, w3, b3, bn, w4, bl4.reshape(1, 1))


def kernel(x, edge_index, batch, ctrl, W_pre, b_pre, gin_params,
           Wl1, bl1, Wl2, bl2, Wl3, bl3, Wl4, bl4,
           bn_g, bn_b, bn_rm, bn_rv):
    npad_e = _EPAD - _E
    pidx = jnp.arange(npad_e, dtype=jnp.int32)
    src_pad = pidx % _N
    src2 = jnp.concatenate(
        [edge_index[0], src_pad]).reshape(_NCHUNKS, _CHUNK)
    dst_seg = jnp.concatenate(
        [edge_index[1], _N + pidx % (_NPAD - _N)]).reshape(_NCHUNKS, _CHUNK)
    dst_dir = jnp.concatenate(
        [edge_index[1], src_pad]).reshape(_NCHUNKS, _CHUNK)
    zeros = jnp.zeros((_ROWS_PER_TILE, _H), jnp.float32)

    h = _pre_call(x, W_pre, b_pre.reshape(1, _H))
    for p in gin_params:
        agg = _seg_sum(h, src2, dst_seg, zeros)
        h = _gin_call(h, agg, p)

    parts = _dirichlet(h, src2, dst_dir)
    o, de = _head_call(ctrl, h, parts.reshape(4, _H),
                       Wl1, bl1, Wl2, bl2, Wl3, bl3, Wl4, bl4,
                       bn_g, bn_b, bn_rm, bn_rv)
    return (o, o, de[0, 0])
